# baseline (speedup 1.0000x reference)
import functools

import jax
import jax.numpy as jnp
from jax import lax
from jax.experimental import pallas as pl
from jax.experimental.pallas import tpu as pltpu
from jax.experimental.pallas import tpu_sc as plsc

_B = 4
_S = 2048
_D = 4096
_L = 16


def _sc_body(hidden_hbm, ids_hbm, out_hbm, ids_v, row_v):
    nc = 2
    wid = lax.axis_index("s") * nc + lax.axis_index("c")

    @pl.when(wid < _B)
    def _():
        b = wid
        pltpu.sync_copy(ids_hbm.at[b], ids_v)

        lane = lax.iota(jnp.int32, _L)

        def step(i, acc):
            vals = ids_v[pl.ds(i * _L, _L)]
            pos = i * _L + lane
            key = vals * _S + (_S - 1 - pos)
            return jnp.maximum(acc, key)

        acc = lax.fori_loop(0, _S // _L, step, jnp.full((_L,), -1, jnp.int32))
        best = acc[0]
        for j in range(1, _L):
            best = jnp.maximum(best, acc[j])
        idx = (_S - 1) - (best & (_S - 1))

        pltpu.sync_copy(hidden_hbm.at[b * _S + idx], row_v)
        pltpu.sync_copy(row_v, out_hbm.at[b])


@jax.jit
def kernel(last_hidden_state, input_ids):
    hidden2d = last_hidden_state.reshape(_B * _S, _D)
    run = pl.kernel(
        _sc_body,
        out_type=jax.ShapeDtypeStruct((_B, _D), jnp.float32),
        mesh=plsc.VectorSubcoreMesh(core_axis_name="c", subcore_axis_name="s"),
        scratch_types=[
            pltpu.VMEM((_S,), jnp.int32),
            pltpu.VMEM((_D,), jnp.float32),
        ],
    )
    return run(hidden2d, input_ids)
